# baseline (speedup 1.0000x reference)
import functools

import jax
import jax.numpy as jnp
from jax import lax
from jax.experimental import pallas as pl
from jax.experimental.pallas import tpu as pltpu
from jax.experimental.pallas import tpu_sc as plsc

N = 10000
E = 320000
D = 128
DOUT = 128
NW = 32
NB = 320
NPAD = NW * NB
NBP = 328
TW = 256
BE = 4000
NBLK = E // BE
CAPB = BE + 32
RB_PRE = 2560
RB_POST = 2000
EPS = 1e-5


def _pre_body(x_ref, w_ref, g_ref, bc_ref, t_ref):
    y = lax.dot_general(x_ref[...], w_ref[...], (((1,), (1,)), ((), ())),
                        preferred_element_type=jnp.float32)
    my = jnp.mean(y, axis=1, keepdims=True)
    yc = y - my
    vy = jnp.mean(yc * yc, axis=1, keepdims=True)
    cy = jnp.mean(yc * bc_ref[...], axis=1, keepdims=True)
    rn = lax.rsqrt(vy + EPS)
    pad = jnp.zeros((yc.shape[0], TW - D - 3), jnp.float32)
    t_ref[...] = jnp.concatenate([yc * g_ref[...], vy, cy, rn, pad], axis=1)


_pre_call = pl.pallas_call(
    _pre_body,
    grid=(NPAD // RB_PRE,),
    in_specs=[
        pl.BlockSpec((RB_PRE, D), lambda i: (i, 0)),
        pl.BlockSpec((D, D), lambda i: (0, 0)),
        pl.BlockSpec((1, D), lambda i: (0, 0)),
        pl.BlockSpec((1, D), lambda i: (0, 0)),
    ],
    out_specs=pl.BlockSpec((RB_PRE, TW), lambda i: (i, 0)),
    out_shape=jax.ShapeDtypeStruct((NPAD, TW), jnp.float32),
)


def _post_body(x_ref, a_ref, wx_ref, wa_ref, fb_ref, fg_ref, flb_ref, o_ref):
    h = (lax.dot_general(x_ref[...], wx_ref[...], (((1,), (1,)), ((), ())),
                         preferred_element_type=jnp.float32)
         + lax.dot_general(a_ref[...], wa_ref[...], (((1,), (1,)), ((), ())),
                           preferred_element_type=jnp.float32)
         + fb_ref[...])
    mu = jnp.mean(h, axis=1, keepdims=True)
    hc = h - mu
    var = jnp.mean(hc * hc, axis=1, keepdims=True)
    o = hc * lax.rsqrt(var + EPS) * fg_ref[...] + flb_ref[...]
    o_ref[...] = jnp.maximum(o, 0.0)


_post_call = pl.pallas_call(
    _post_body,
    grid=(N // RB_POST,),
    in_specs=[
        pl.BlockSpec((RB_POST, D), lambda i: (i, 0)),
        pl.BlockSpec((RB_POST, D), lambda i: (i, 0)),
        pl.BlockSpec((D, D), lambda i: (0, 0)),
        pl.BlockSpec((D, D), lambda i: (0, 0)),
        pl.BlockSpec((1, DOUT), lambda i: (0, 0)),
        pl.BlockSpec((1, DOUT), lambda i: (0, 0)),
        pl.BlockSpec((1, DOUT), lambda i: (0, 0)),
    ],
    out_specs=pl.BlockSpec((RB_POST, DOUT), lambda i: (i, 0)),
    out_shape=jax.ShapeDtypeStruct((N, DOUT), jnp.float32),
)


@functools.partial(
    pl.kernel,
    out_type=jax.ShapeDtypeStruct((NPAD, D), jnp.float32),
    mesh=plsc.VectorSubcoreMesh(core_axis_name="c", subcore_axis_name="s"),
    scratch_types=[
        pltpu.VMEM((BE,), jnp.int32),
        pltpu.VMEM((BE,), jnp.int32),
        pltpu.VMEM((BE,), jnp.float32),
        pltpu.VMEM((CAPB,), jnp.int32),
        pltpu.VMEM((CAPB,), jnp.int32),
        pltpu.VMEM((CAPB,), jnp.float32),
        pltpu.VMEM((16, TW), jnp.float32),
        pltpu.VMEM((NBP, D), jnp.float32),
        pltpu.VMEM((19, 16), jnp.float32),
        pltpu.VMEM((40,), jnp.int32),
        pltpu.VMEM((40,), jnp.int32),
        pltpu.VMEM((40,), jnp.int32),
        pltpu.VMEM((40,), jnp.int32),
        pltpu.VMEM((40,), jnp.float32),
        pltpu.SemaphoreType.DMA,
    ],
)
def _sc_agg(t_hbm, src_hbm, dst_hbm, w_hbm, p_hbm, out_hbm,
            dstv, srcv, wv, locb, srcb, wb, rows, acc, pv,
            tmx, tmn, tl, ts, tw, sem):
    cid = lax.axis_index("c")
    sid = lax.axis_index("s")
    wid = sid * 2 + cid
    base = wid * NB

    pltpu.sync_copy(p_hbm, pv)

    zero = jnp.zeros((16,), jnp.float32)
    zero_i = jnp.zeros((16,), jnp.int32)

    def _zero_acc(i, carry):
        for j in range(D // 16):
            acc[i, pl.ds(16 * j, 16)] = zero
        return carry

    lax.fori_loop(0, NBP, _zero_acc, 0)

    tmx[pl.ds(0, 16)] = zero_i
    tmn[pl.ds(0, 16)] = jnp.full((16,), 127, jnp.int32)

    vb_s = pv[16][0]
    c_s = pv[17][0]
    rb_s = pv[18][0]
    iota16 = lax.iota(jnp.int32, 16)
    dummy_loc = jnp.full((16,), NB, jnp.int32)

    def _lane_sum(v):
        for k in (1, 2, 4, 8):
            tmx[pl.ds(8, 16)] = v
            v = v + tmx[pl.ds(8 - k, 16)]
        return v[15]

    def _lane_min(v):
        for k in (1, 2, 4, 8):
            tmn[pl.ds(8, 16)] = v
            v = jnp.minimum(v, tmn[pl.ds(8 - k, 16)])
        return v[15]

    def _process_group(off):
        src16 = srcb.at[pl.ds(off, 16)]
        w16 = wb[pl.ds(off, 16)]
        loc16 = locb[pl.ds(off, 16)]
        pltpu.async_copy(t_hbm.at[src16], rows, sem).wait()
        for e in range(16):
            stat = rows[e, pl.ds(D, 16)]
            vy_s = stat[0]
            cy_s = stat[1]
            rn_s = stat[2]
            w_s = w16[e]
            dloc = loc16[e]
            s_s = 1.0 + c_s * w_s
            varp = s_s * (s_s * vy_s + 2.0 * cy_s) + vb_s + EPS
            r = 0.6 * jnp.minimum(rn_s * (1.457 - 0.5 * s_s), rb_s)
            for _ in range(7):
                r = r * (1.5 - 0.5 * varp * r * r)
            u_s = s_s * r
            v_s = r
            for j in range(D // 16):
                gvec = rows[e, pl.ds(16 * j, 16)]
                hv = v_s * pv[j] + pv[8 + j]
                o = jnp.maximum(u_s * gvec + hv, 0.0)
                av = acc[dloc, pl.ds(16 * j, 16)]
                acc[dloc, pl.ds(16 * j, 16)] = jnp.maximum(av, o)

    def _block(b, carry):
        e0 = b * BE
        pltpu.sync_copy(dst_hbm.at[pl.ds(e0, BE)], dstv)
        pltpu.sync_copy(src_hbm.at[pl.ds(e0, BE)], srcv)
        pltpu.sync_copy(w_hbm.at[pl.ds(e0, BE)], wv)

        def _scan(i, cnt):
            loc = dstv[pl.ds(i * 16, 16)] - base
            m = 1 - lax.shift_right_logical(loc | (NB - 1 - loc), 31)
            nm = _lane_sum(m)
            tl[pl.ds(8, 16)] = loc
            ts[pl.ds(8, 16)] = srcv[pl.ds(i * 16, 16)]
            tw[pl.ds(8, 16)] = wv[pl.ds(i * 16, 16)]

            def _take(t, st):
                mm, cn = st
                f = _lane_min(iota16 + (1 - mm) * 99)
                locb[pl.ds(cn, 16)] = tl[pl.ds(8 + f, 16)]
                srcb[pl.ds(cn, 16)] = ts[pl.ds(8 + f, 16)]
                wb[pl.ds(cn, 16)] = tw[pl.ds(8 + f, 16)]
                dd = iota16 - f
                iszero = 1 - lax.shift_right_logical(dd | (0 - dd), 31)
                return mm * (1 - iszero), cn + 1

            return lax.fori_loop(0, nm, _take, (m, cnt))[1]

        cnt = lax.fori_loop(0, BE // 16, _scan, jnp.int32(0))

        locb[pl.ds(cnt, 16)] = dummy_loc
        srcb[pl.ds(cnt, 16)] = zero_i
        wb[pl.ds(cnt, 16)] = zero

        def _group(g, carry2):
            _process_group(g * 16)
            return carry2

        lax.fori_loop(0, lax.shift_right_logical(cnt + 15, 4), _group, 0)
        return carry

    lax.fori_loop(0, NBLK, _block, 0)
    pltpu.sync_copy(acc.at[pl.ds(0, NB)], out_hbm.at[pl.ds(base, NB)])


def kernel(x, edge_index, edge_weight, pool_W, pool_b, ln_pool_g, ln_pool_b,
           final_W, final_b, ln_final_g, ln_final_b, coeff):
    src = edge_index[0].astype(jnp.int32)
    dst = edge_index[1].astype(jnp.int32)
    w = edge_weight.astype(jnp.float32)

    xp = jnp.pad(x, ((0, NPAD - N), (0, 0)))
    bc = (pool_b - jnp.mean(pool_b)).reshape(1, D)
    table = _pre_call(xp, pool_W, ln_pool_g.reshape(1, D), bc)

    c = jax.nn.softplus(coeff)
    h_vec = (pool_b - jnp.mean(pool_b)) * ln_pool_g
    vb = jnp.mean((pool_b - jnp.mean(pool_b)) ** 2)
    rb = lax.rsqrt(vb + EPS)
    params = jnp.concatenate([
        h_vec.reshape(8, 16),
        ln_pool_b.reshape(8, 16),
        jnp.full((1, 16), vb, jnp.float32),
        jnp.full((1, 16), c, jnp.float32),
        jnp.full((1, 16), rb, jnp.float32),
    ], axis=0)

    agg = _sc_agg(table, src, dst, w, params)[:N]

    return _post_call(x, agg, final_W[:, :D], final_W[:, D:],
                      final_b.reshape(1, DOUT), ln_final_g.reshape(1, DOUT),
                      ln_final_b.reshape(1, DOUT))
